# Initial kernel scaffold; baseline (speedup 1.0000x reference)
import jax
import jax.numpy as jnp
from jax.experimental import pallas as pl


def kernel(x, edge_index, stochastic_feature, W_sgc, b_sgc, W_last, b_last):
    raise NotImplementedError("write your pallas kernel here")


# speedup vs baseline: 9.2273x; 9.2273x over previous
import functools

import jax
import jax.numpy as jnp
from jax import lax
from jax.experimental import pallas as pl
from jax.experimental.pallas import tpu as pltpu
from jax.experimental.pallas import tpu_sc as plsc

N_NODES = 10000
FEAT = 128
E_EDGES = 320000
NC = 2
NS = 16
CHUNK = 128
E_PAD = 323584
CPT16 = E_PAD // NS // CHUNK
CPT32 = E_PAD // (NC * NS) // CHUNK
NP = 10112
RPT = NP // NS
BN = 2000

_mesh = plsc.VectorSubcoreMesh(core_axis_name="c", subcore_axis_name="s")


def _deg_body(rowp, ones128, zeros128, out0, out1, ridx, buf, acc):
    c = lax.axis_index("c")
    s = lax.axis_index("s")
    wb = s * RPT
    pltpu.sync_copy(zeros128, buf)
    for k in range(4):
        pltpu.sync_copy(buf, acc.at[pl.ds(wb + k * CHUNK, CHUNK)])
    pltpu.sync_copy(buf.at[pl.ds(0, RPT - 4 * CHUNK)],
                    acc.at[pl.ds(wb + 4 * CHUNK, RPT - 4 * CHUNK)])
    plsc.subcore_barrier()
    pltpu.sync_copy(ones128, buf)
    wid = s * NC + c

    def body(i, carry):
        eb = wid * (CPT32 * CHUNK) + i * CHUNK
        pltpu.sync_copy(rowp.at[pl.ds(eb, CHUNK)], ridx)
        pltpu.sync_copy(buf, acc.at[ridx], add=True)
        return carry

    lax.fori_loop(0, CPT32, body, 0)
    plsc.subcore_barrier()

    @pl.when(c == 0)
    def _():
        pltpu.sync_copy(acc.at[pl.ds(wb, RPT)], out0.at[pl.ds(wb, RPT)])

    @pl.when(c == 1)
    def _():
        pltpu.sync_copy(acc.at[pl.ds(wb, RPT)], out1.at[pl.ds(wb, RPT)])


def _spmm_body(hx, hn, rowp, colp, zeros128, outx, outn,
               cidx, ridx, rows, acc, sem):
    c = lax.axis_index("c")
    s = lax.axis_index("s")
    wb = s * RPT
    pltpu.sync_copy(zeros128, rows)
    for k in range(4):
        pltpu.sync_copy(rows, acc.at[pl.ds(wb + k * CHUNK, CHUNK)])
    pltpu.sync_copy(rows.at[pl.ds(0, RPT - 4 * CHUNK)],
                    acc.at[pl.ds(wb + 4 * CHUNK, RPT - 4 * CHUNK)])
    plsc.subcore_barrier()

    def run(h_hbm):
        def body(i, carry):
            eb = s * (CPT16 * CHUNK) + i * CHUNK
            pltpu.sync_copy(colp.at[pl.ds(eb, CHUNK)], cidx)
            pltpu.sync_copy(rowp.at[pl.ds(eb, CHUNK)], ridx)
            pltpu.async_copy(h_hbm.at[cidx], rows, sem).wait()
            pltpu.sync_copy(rows, acc.at[ridx], add=True)
            return carry

        lax.fori_loop(0, CPT16, body, 0)

    @pl.when(c == 0)
    def _():
        run(hx)

    @pl.when(c == 1)
    def _():
        run(hn)

    plsc.subcore_barrier()

    @pl.when(c == 0)
    def _():
        pltpu.sync_copy(acc.at[pl.ds(wb, RPT)], outx.at[pl.ds(wb, RPT)])

    @pl.when(c == 1)
    def _():
        pltpu.sync_copy(acc.at[pl.ds(wb, RPT)], outn.at[pl.ds(wb, RPT)])


def _make_deg_kernel(interpret=False):
    return pl.kernel(
        _deg_body,
        out_type=(jax.ShapeDtypeStruct((NP, FEAT), jnp.float32),
                  jax.ShapeDtypeStruct((NP, FEAT), jnp.float32)),
        mesh=_mesh,
        scratch_types=[
            pltpu.VMEM((CHUNK,), jnp.int32),
            pltpu.VMEM((CHUNK, FEAT), jnp.float32),
            pltpu.VMEM_SHARED((NP, FEAT), jnp.float32),
        ],
        interpret=interpret,
    )


def _make_spmm_kernel(interpret=False):
    return pl.kernel(
        _spmm_body,
        out_type=(jax.ShapeDtypeStruct((NP, FEAT), jnp.float32),
                  jax.ShapeDtypeStruct((NP, FEAT), jnp.float32)),
        mesh=_mesh,
        scratch_types=[
            pltpu.VMEM((CHUNK,), jnp.int32),
            pltpu.VMEM((CHUNK,), jnp.int32),
            pltpu.VMEM((CHUNK, FEAT), jnp.float32),
            pltpu.VMEM_SHARED((NP, FEAT), jnp.float32),
            pltpu.SemaphoreType.DMA,
        ],
        interpret=interpret,
    )


_deg_kernel = _make_deg_kernel()
_spmm_kernel = _make_spmm_kernel()


def _k1_body(d0, d1, x, sf, ox, on):
    deg = d0[:, 0:1] + d1[:, 0:1] + 1.0
    dinv = lax.rsqrt(deg)
    ox[...] = x[...] * dinv
    on[...] = sf[...] * dinv


def _k2_body(d0, d1, s1x, s1n, h1x, h1n, ox, on):
    deg = d0[:, 0:1] + d1[:, 0:1] + 1.0
    ox[...] = (s1x[...] + h1x[...]) / deg
    on[...] = (s1n[...] + h1n[...]) / deg


def _k3_body(d0, d1, s2x, s2n, h2x, h2n, wsgc, bsgc, wl1, wl2, bl, out):
    deg = d0[:, 0:1] + d1[:, 0:1] + 1.0
    dinv = lax.rsqrt(deg)
    hx = (s2x[...] + h2x[...]) * dinv
    noise = (s2n[...] + h2n[...]) * dinv
    z = jnp.dot(hx, wsgc[...], preferred_element_type=jnp.float32) + bsgc[...]
    nrm = jnp.sqrt(jnp.sum(z * z, axis=-1, keepdims=True))
    z = z / jnp.maximum(nrm, 1e-12)
    out[...] = (jnp.dot(z, wl1[...], preferred_element_type=jnp.float32)
                + jnp.dot(noise, wl2[...], preferred_element_type=jnp.float32)
                + bl[...])


def _row_spec(w):
    return pl.BlockSpec((BN, w), lambda i: (i, 0))


def _full_spec(r, w):
    return pl.BlockSpec((r, w), lambda i: (0, 0))


_GRID = N_NODES // BN

_k1 = pl.pallas_call(
    _k1_body,
    grid=(_GRID,),
    in_specs=[_row_spec(FEAT), _row_spec(FEAT), _row_spec(FEAT), _row_spec(FEAT)],
    out_specs=(_row_spec(FEAT), _row_spec(FEAT)),
    out_shape=(jax.ShapeDtypeStruct((N_NODES, FEAT), jnp.float32),
               jax.ShapeDtypeStruct((N_NODES, FEAT), jnp.float32)),
)

_k2 = pl.pallas_call(
    _k2_body,
    grid=(_GRID,),
    in_specs=[_row_spec(FEAT), _row_spec(FEAT),
              _row_spec(FEAT), _row_spec(FEAT), _row_spec(FEAT), _row_spec(FEAT)],
    out_specs=(_row_spec(FEAT), _row_spec(FEAT)),
    out_shape=(jax.ShapeDtypeStruct((N_NODES, FEAT), jnp.float32),
               jax.ShapeDtypeStruct((N_NODES, FEAT), jnp.float32)),
)

_k3 = pl.pallas_call(
    _k3_body,
    grid=(_GRID,),
    in_specs=[_row_spec(FEAT), _row_spec(FEAT),
              _row_spec(FEAT), _row_spec(FEAT), _row_spec(FEAT), _row_spec(FEAT),
              _full_spec(FEAT, FEAT), _full_spec(1, FEAT),
              _full_spec(FEAT, FEAT), _full_spec(FEAT, FEAT), _full_spec(1, FEAT)],
    out_specs=pl.BlockSpec((BN, FEAT), lambda i: (i, 0)),
    out_shape=jax.ShapeDtypeStruct((N_NODES, FEAT), jnp.float32),
)


@jax.jit
def kernel(x, edge_index, stochastic_feature, W_sgc, b_sgc, W_last, b_last):
    row = edge_index[0].astype(jnp.int32)
    col = edge_index[1].astype(jnp.int32)
    pad = E_PAD - E_EDGES
    rowp = jnp.concatenate([row, jnp.full((pad,), N_NODES, jnp.int32)])
    colp = jnp.concatenate([col, jnp.zeros((pad,), jnp.int32)])
    ones128 = jnp.ones((CHUNK, FEAT), jnp.float32)
    zeros128 = jnp.zeros((CHUNK, FEAT), jnp.float32)

    d0, d1 = _deg_kernel(rowp, ones128, zeros128)

    h1x, h1n = _k1(d0[:N_NODES], d1[:N_NODES], x, stochastic_feature)
    s1x, s1n = _spmm_kernel(h1x, h1n, rowp, colp, zeros128)
    h2x, h2n = _k2(d0[:N_NODES], d1[:N_NODES],
                   s1x[:N_NODES], s1n[:N_NODES], h1x, h1n)
    s2x, s2n = _spmm_kernel(h2x, h2n, rowp, colp, zeros128)
    out = _k3(d0[:N_NODES], d1[:N_NODES],
              s2x[:N_NODES], s2n[:N_NODES], h2x, h2n,
              W_sgc, b_sgc.reshape(1, FEAT),
              W_last[:FEAT], W_last[FEAT:], b_last.reshape(1, FEAT))
    return out


# speedup vs baseline: 10.3044x; 1.1167x over previous
import functools

import jax
import jax.numpy as jnp
from jax import lax
from jax.experimental import pallas as pl
from jax.experimental.pallas import tpu as pltpu
from jax.experimental.pallas import tpu_sc as plsc

N_NODES = 10000
FEAT = 128
E_EDGES = 320000
NC = 2
NS = 16
CHUNK = 128
E_PAD = 327680
CPT16 = E_PAD // NS // CHUNK
CPT32 = E_PAD // (NC * NS) // CHUNK
IB_SP = 32
IB_DG = 16
NP = 10112
RPT = NP // NS
BN = 2000

_mesh = plsc.VectorSubcoreMesh(core_axis_name="c", subcore_axis_name="s")


def _deg_body(row2d, ones128, zeros128, out0, out1, rix, buf, acc):
    c = lax.axis_index("c")
    s = lax.axis_index("s")
    wb = s * RPT
    pltpu.sync_copy(zeros128, buf)
    for k in range(4):
        pltpu.sync_copy(buf, acc.at[pl.ds(wb + k * CHUNK, CHUNK)])
    pltpu.sync_copy(buf.at[pl.ds(0, RPT - 4 * CHUNK)],
                    acc.at[pl.ds(wb + 4 * CHUNK, RPT - 4 * CHUNK)])
    pltpu.sync_copy(ones128, buf)
    wid = s * NC + c
    plsc.subcore_barrier()

    def blk(b, carry):
        pltpu.sync_copy(row2d.at[pl.ds(wid * CPT32 + b * IB_DG, IB_DG)], rix)

        def body(i, c2):
            pltpu.sync_copy(buf, acc.at[rix.at[i]], add=True)
            return c2

        lax.fori_loop(0, IB_DG, body, 0)
        return carry

    lax.fori_loop(0, CPT32 // IB_DG, blk, 0)
    plsc.subcore_barrier()

    @pl.when(c == 0)
    def _():
        pltpu.sync_copy(acc.at[pl.ds(wb, RPT)], out0.at[pl.ds(wb, RPT)])

    @pl.when(c == 1)
    def _():
        pltpu.sync_copy(acc.at[pl.ds(wb, RPT)], out1.at[pl.ds(wb, RPT)])


def _spmm_body(hx, hn, row2d, col2d, zeros128, outx, outn,
               cix, rix, rows0, rows1, acc, sem0, sem1):
    c = lax.axis_index("c")
    s = lax.axis_index("s")
    wb = s * RPT
    pltpu.sync_copy(zeros128, rows0)
    for k in range(4):
        pltpu.sync_copy(rows0, acc.at[pl.ds(wb + k * CHUNK, CHUNK)])
    pltpu.sync_copy(rows0.at[pl.ds(0, RPT - 4 * CHUNK)],
                    acc.at[pl.ds(wb + 4 * CHUNK, RPT - 4 * CHUNK)])
    plsc.subcore_barrier()

    def run(h_hbm):
        def blk(b, carry):
            base = s * CPT16 + b * IB_SP
            pltpu.sync_copy(col2d.at[pl.ds(base, IB_SP)], cix)
            pltpu.sync_copy(row2d.at[pl.ds(base, IB_SP)], rix)
            pltpu.async_copy(h_hbm.at[cix.at[0]], rows0, sem0)

            def body(j, c2):
                i0 = 2 * j
                pltpu.async_copy(h_hbm.at[cix.at[i0 + 1]], rows1, sem1)
                pltpu.make_async_copy(h_hbm.at[cix.at[0]], rows0, sem0).wait()
                pltpu.sync_copy(rows0, acc.at[rix.at[i0]], add=True)

                @pl.when(j < IB_SP // 2 - 1)
                def _():
                    pltpu.async_copy(h_hbm.at[cix.at[i0 + 2]], rows0, sem0)

                pltpu.make_async_copy(h_hbm.at[cix.at[0]], rows1, sem1).wait()
                pltpu.sync_copy(rows1, acc.at[rix.at[i0 + 1]], add=True)
                return c2

            lax.fori_loop(0, IB_SP // 2, body, 0)
            return carry

        lax.fori_loop(0, CPT16 // IB_SP, blk, 0)

    @pl.when(c == 0)
    def _():
        run(hx)

    @pl.when(c == 1)
    def _():
        run(hn)

    plsc.subcore_barrier()

    @pl.when(c == 0)
    def _():
        pltpu.sync_copy(acc.at[pl.ds(wb, RPT)], outx.at[pl.ds(wb, RPT)])

    @pl.when(c == 1)
    def _():
        pltpu.sync_copy(acc.at[pl.ds(wb, RPT)], outn.at[pl.ds(wb, RPT)])


def _make_deg_kernel(interpret=False):
    return pl.kernel(
        _deg_body,
        out_type=(jax.ShapeDtypeStruct((NP, FEAT), jnp.float32),
                  jax.ShapeDtypeStruct((NP, FEAT), jnp.float32)),
        mesh=_mesh,
        scratch_types=[
            pltpu.VMEM((IB_DG, CHUNK), jnp.int32),
            pltpu.VMEM((CHUNK, FEAT), jnp.float32),
            pltpu.VMEM_SHARED((NP, FEAT), jnp.float32),
        ],
        interpret=interpret,
    )


def _make_spmm_kernel(interpret=False):
    return pl.kernel(
        _spmm_body,
        out_type=(jax.ShapeDtypeStruct((NP, FEAT), jnp.float32),
                  jax.ShapeDtypeStruct((NP, FEAT), jnp.float32)),
        mesh=_mesh,
        scratch_types=[
            pltpu.VMEM((IB_SP, CHUNK), jnp.int32),
            pltpu.VMEM((IB_SP, CHUNK), jnp.int32),
            pltpu.VMEM((CHUNK, FEAT), jnp.float32),
            pltpu.VMEM((CHUNK, FEAT), jnp.float32),
            pltpu.VMEM_SHARED((NP, FEAT), jnp.float32),
            pltpu.SemaphoreType.DMA,
            pltpu.SemaphoreType.DMA,
        ],
        interpret=interpret,
    )


_deg_kernel = _make_deg_kernel()
_spmm_kernel = _make_spmm_kernel()


def _k1_body(d0, d1, x, sf, ox, on):
    deg = d0[:, 0:1] + d1[:, 0:1] + 1.0
    dinv = lax.rsqrt(deg)
    ox[...] = x[...] * dinv
    on[...] = sf[...] * dinv


def _k2_body(d0, d1, s1x, s1n, h1x, h1n, ox, on):
    deg = d0[:, 0:1] + d1[:, 0:1] + 1.0
    ox[...] = (s1x[...] + h1x[...]) / deg
    on[...] = (s1n[...] + h1n[...]) / deg


def _k3_body(d0, d1, s2x, s2n, h2x, h2n, wsgc, bsgc, wl1, wl2, bl, out):
    deg = d0[:, 0:1] + d1[:, 0:1] + 1.0
    dinv = lax.rsqrt(deg)
    hx = (s2x[...] + h2x[...]) * dinv
    noise = (s2n[...] + h2n[...]) * dinv
    z = jnp.dot(hx, wsgc[...], preferred_element_type=jnp.float32) + bsgc[...]
    nrm = jnp.sqrt(jnp.sum(z * z, axis=-1, keepdims=True))
    z = z / jnp.maximum(nrm, 1e-12)
    out[...] = (jnp.dot(z, wl1[...], preferred_element_type=jnp.float32)
                + jnp.dot(noise, wl2[...], preferred_element_type=jnp.float32)
                + bl[...])


def _row_spec(w):
    return pl.BlockSpec((BN, w), lambda i: (i, 0))


def _full_spec(r, w):
    return pl.BlockSpec((r, w), lambda i: (0, 0))


_GRID = N_NODES // BN

_k1 = pl.pallas_call(
    _k1_body,
    grid=(_GRID,),
    in_specs=[_row_spec(FEAT), _row_spec(FEAT), _row_spec(FEAT), _row_spec(FEAT)],
    out_specs=(_row_spec(FEAT), _row_spec(FEAT)),
    out_shape=(jax.ShapeDtypeStruct((N_NODES, FEAT), jnp.float32),
               jax.ShapeDtypeStruct((N_NODES, FEAT), jnp.float32)),
)

_k2 = pl.pallas_call(
    _k2_body,
    grid=(_GRID,),
    in_specs=[_row_spec(FEAT), _row_spec(FEAT),
              _row_spec(FEAT), _row_spec(FEAT), _row_spec(FEAT), _row_spec(FEAT)],
    out_specs=(_row_spec(FEAT), _row_spec(FEAT)),
    out_shape=(jax.ShapeDtypeStruct((N_NODES, FEAT), jnp.float32),
               jax.ShapeDtypeStruct((N_NODES, FEAT), jnp.float32)),
)

_k3 = pl.pallas_call(
    _k3_body,
    grid=(_GRID,),
    in_specs=[_row_spec(FEAT), _row_spec(FEAT),
              _row_spec(FEAT), _row_spec(FEAT), _row_spec(FEAT), _row_spec(FEAT),
              _full_spec(FEAT, FEAT), _full_spec(1, FEAT),
              _full_spec(FEAT, FEAT), _full_spec(FEAT, FEAT), _full_spec(1, FEAT)],
    out_specs=pl.BlockSpec((BN, FEAT), lambda i: (i, 0)),
    out_shape=jax.ShapeDtypeStruct((N_NODES, FEAT), jnp.float32),
)


@jax.jit
def kernel(x, edge_index, stochastic_feature, W_sgc, b_sgc, W_last, b_last):
    row = edge_index[0].astype(jnp.int32)
    col = edge_index[1].astype(jnp.int32)
    pad = E_PAD - E_EDGES
    rowp = jnp.concatenate([row, jnp.full((pad,), N_NODES, jnp.int32)])
    colp = jnp.concatenate([col, jnp.zeros((pad,), jnp.int32)])
    row2d = rowp.reshape(E_PAD // CHUNK, CHUNK)
    col2d = colp.reshape(E_PAD // CHUNK, CHUNK)
    ones128 = jnp.ones((CHUNK, FEAT), jnp.float32)
    zeros128 = jnp.zeros((CHUNK, FEAT), jnp.float32)

    d0, d1 = _deg_kernel(row2d, ones128, zeros128)

    h1x, h1n = _k1(d0[:N_NODES], d1[:N_NODES], x, stochastic_feature)
    s1x, s1n = _spmm_kernel(h1x, h1n, row2d, col2d, zeros128)
    h2x, h2n = _k2(d0[:N_NODES], d1[:N_NODES],
                   s1x[:N_NODES], s1n[:N_NODES], h1x, h1n)
    s2x, s2n = _spmm_kernel(h2x, h2n, row2d, col2d, zeros128)
    out = _k3(d0[:N_NODES], d1[:N_NODES],
              s2x[:N_NODES], s2n[:N_NODES], h2x, h2n,
              W_sgc, b_sgc.reshape(1, FEAT),
              W_last[:FEAT], W_last[FEAT:], b_last.reshape(1, FEAT))
    return out
